# baseline (speedup 1.0000x reference)
import functools

import jax
import jax.numpy as jnp
from jax import lax
from jax.experimental import pallas as pl
from jax.experimental.pallas import tpu as pltpu
from jax.experimental.pallas import tpu_sc as plsc

_H = 768
_F = 1024
_E = 64
_T = 2048
_CAP = 256
_CAPP = 264
_NROWS = _E * _CAPP
_AUX_COEF = 0.01
_Z_COEF = 0.001
_NW = 32
_TPW = _T // _NW


def _router_body(x_ref, wr_ref, se_ref, so_ref, we_ref, wo_ref, loss_ref):
    x = x_ref[...]
    wr = wr_ref[...]
    logits = lax.dot_general(x, wr, (((1,), (1,)), ((), ())),
                             preferred_element_type=jnp.float32)
    l3 = logits.reshape(16, 128, _E)
    m3 = jnp.max(l3, axis=2, keepdims=True)
    el = jnp.exp(l3 - m3)
    s3 = jnp.sum(el, axis=2, keepdims=True)
    probs = el / s3
    lse = m3[:, :, 0] + jnp.log(s3[:, :, 0])
    z_loss = _Z_COEF * jnp.mean(lse * lse)

    iota3 = lax.broadcasted_iota(jnp.int32, (16, 128, _E), 2)
    v1 = jnp.max(probs, axis=2, keepdims=True)
    i1 = jnp.min(jnp.where(probs == v1, iota3, _E), axis=2)
    oh1 = (iota3 == i1[:, :, None]).astype(jnp.float32)
    probs_m = jnp.where(oh1 > 0.0, -1.0, probs)
    v2 = jnp.max(probs_m, axis=2, keepdims=True)
    i2 = jnp.min(jnp.where(probs_m == v2, iota3, _E), axis=2)
    oh2 = (iota3 == i2[:, :, None]).astype(jnp.float32)

    denom = v1[:, :, 0] + v2[:, :, 0]
    w1 = v1[:, :, 0] / denom
    w2 = v2[:, :, 0] / denom

    oh12 = oh1 + oh2
    counts_e = jnp.sum(oh12, axis=(0, 1))
    f_e = counts_e / (2.0 * _T)
    p_e = jnp.mean(probs, axis=(0, 1))
    aux = _AUX_COEF * _E * jnp.sum(f_e * p_e)
    loss_ref[...] = jnp.broadcast_to(aux + z_loss, (1, 1))

    rr = lax.broadcasted_iota(jnp.int32, (128, 128), 0)
    cc = lax.broadcasted_iota(jnp.int32, (128, 128), 1)
    tril = (rr > cc).astype(jnp.float32)
    rr16 = lax.broadcasted_iota(jnp.int32, (16, 16), 0)
    cc16 = lax.broadcasted_iota(jnp.int32, (16, 16), 1)
    tril16 = (rr16 > cc16).astype(jnp.float32)
    bsums = jnp.sum(oh12, axis=1)
    offs = lax.dot_general(tril16, bsums, (((1,), (0,)), ((), ())),
                           preferred_element_type=jnp.float32)
    pos0_l, pos1_l = [], []
    for b in range(16):
        cx = lax.dot_general(tril, oh12[b], (((1,), (0,)), ((), ())),
                             preferred_element_type=jnp.float32)
        cx = cx + offs[b][None, :]
        pos0_l.append(jnp.sum(cx * oh1[b], axis=1))
        pos1_l.append(jnp.sum(cx * oh2[b], axis=1))
    pos0 = jnp.stack(pos0_l)
    pos1 = jnp.stack(pos1_l)
    keep0 = (pos0 < float(_CAP)).astype(jnp.float32)
    keep1 = (pos1 < float(_CAP)).astype(jnp.float32)
    slot0 = jnp.minimum(pos0, float(_CAP)).astype(jnp.int32)
    slot1 = jnp.minimum(pos1, float(_CAP)).astype(jnp.int32)
    se_ref[...] = i1 * _CAPP + slot0
    so_ref[...] = i2 * _CAPP + slot1
    we_ref[...] = jnp.broadcast_to((w1 * keep0)[:, :, None], (16, 128, 16))
    wo_ref[...] = jnp.broadcast_to((w2 * keep1)[:, :, None], (16, 128, 16))


def _router(x, w_router):
    return pl.pallas_call(
        _router_body,
        out_shape=(
            jax.ShapeDtypeStruct((16, 128), jnp.int32),
            jax.ShapeDtypeStruct((16, 128), jnp.int32),
            jax.ShapeDtypeStruct((16, 128, 16), jnp.float32),
            jax.ShapeDtypeStruct((16, 128, 16), jnp.float32),
            jax.ShapeDtypeStruct((1, 1), jnp.float32),
        ),
    )(x, w_router)


def _dispatch(x, se, so):
    mesh = plsc.VectorSubcoreMesh(core_axis_name="c", subcore_axis_name="s",
                                  num_cores=2, num_subcores=16)

    @functools.partial(
        pl.kernel,
        mesh=mesh,
        out_type=jax.ShapeDtypeStruct((_NROWS, _H), jnp.float32),
        scratch_types=[
            pltpu.VMEM((_TPW,), jnp.int32),
            pltpu.VMEM((_TPW,), jnp.int32),
            pltpu.VMEM((_TPW, _H), jnp.float32),
            pltpu.SemaphoreType.DMA,
        ],
    )
    def k(x_hbm, se_hbm, so_hbm, buf_hbm, idx_e, idx_o, rows, sem):
        wid = lax.axis_index("s") * 2 + lax.axis_index("c")
        base = wid * _TPW
        pltpu.sync_copy(x_hbm.at[pl.ds(base, _TPW)], rows)
        pltpu.sync_copy(se_hbm.at[pl.ds(base, _TPW)], idx_e)
        pltpu.sync_copy(so_hbm.at[pl.ds(base, _TPW)], idx_o)
        pltpu.async_copy(rows, buf_hbm.at[idx_e], sem).wait()
        pltpu.async_copy(rows, buf_hbm.at[idx_o], sem).wait()

    return k(x, se, so)


_FSPLIT = 2
_FH = _F // _FSPLIT


def _mlp_body(buf_ref, wg_ref, wu_ref, wd_ref, out_ref):
    f = pl.program_id(1)
    xb = buf_ref[0].astype(jnp.bfloat16)
    wg = wg_ref[0].astype(jnp.bfloat16)
    wu = wu_ref[0].astype(jnp.bfloat16)
    g = jnp.dot(xb, wg, preferred_element_type=jnp.float32)
    u = jnp.dot(xb, wu, preferred_element_type=jnp.float32)
    h = (g / (1.0 + jnp.exp(-g))) * u
    wd = wd_ref[0].astype(jnp.bfloat16)
    y = jnp.dot(h.astype(jnp.bfloat16), wd,
                preferred_element_type=jnp.float32)

    @pl.when(f == 0)
    def _():
        out_ref[0] = y

    @pl.when(f != 0)
    def _():
        out_ref[0] += y


def _mlp(buf3, w_gate, w_up, w_down):
    return pl.pallas_call(
        _mlp_body,
        grid=(_E, _FSPLIT),
        in_specs=[
            pl.BlockSpec((1, _CAPP, _H), lambda e, f: (e, 0, 0)),
            pl.BlockSpec((1, _H, _FH), lambda e, f: (e, 0, f)),
            pl.BlockSpec((1, _H, _FH), lambda e, f: (e, 0, f)),
            pl.BlockSpec((1, _FH, _H), lambda e, f: (e, f, 0)),
        ],
        out_specs=pl.BlockSpec((1, _CAPP, _H), lambda e, f: (e, 0, 0)),
        out_shape=jax.ShapeDtypeStruct((_E, _CAPP, _H), jnp.float32),
        compiler_params=pltpu.CompilerParams(
            dimension_semantics=("arbitrary", "arbitrary")),
    )(buf3, w_gate, w_up, w_down)


def _combine(eo, se, so, we, wo):
    mesh = plsc.VectorSubcoreMesh(core_axis_name="c", subcore_axis_name="s",
                                  num_cores=2, num_subcores=16)
    @functools.partial(
        pl.kernel,
        mesh=mesh,
        out_type=jax.ShapeDtypeStruct((_T, _H), jnp.float32),
        scratch_types=[
            pltpu.VMEM((_TPW,), jnp.int32),
            pltpu.VMEM((_TPW,), jnp.int32),
            pltpu.VMEM((_TPW, 16), jnp.float32),
            pltpu.VMEM((_TPW, 16), jnp.float32),
            pltpu.VMEM((_TPW, _H), jnp.float32),
            pltpu.VMEM((_TPW, _H), jnp.float32),
            pltpu.SemaphoreType.DMA,
        ],
    )
    def k(eo_hbm, se_hbm, so_hbm, we_hbm, wo_hbm, out_hbm,
          idx_e, idx_o, w0v, w1v, a, b, sem):
        wid = lax.axis_index("s") * 2 + lax.axis_index("c")
        base = wid * _TPW
        pltpu.sync_copy(se_hbm.at[pl.ds(base, _TPW)], idx_e)
        pltpu.sync_copy(so_hbm.at[pl.ds(base, _TPW)], idx_o)
        pltpu.sync_copy(we_hbm.at[pl.ds(base, _TPW)], w0v)
        pltpu.sync_copy(wo_hbm.at[pl.ds(base, _TPW)], w1v)
        pltpu.async_copy(eo_hbm.at[idx_e], a, sem).wait()
        pltpu.async_copy(eo_hbm.at[idx_o], b, sem).wait()

        def row_body(t, carry):
            w0 = w0v[t, :]
            w1 = w1v[t, :]
            for ci in range(_H // 16):
                sl = pl.ds(ci * 16, 16)
                a[t, sl] = a[t, sl] * w0 + b[t, sl] * w1
            return carry

        lax.fori_loop(0, _TPW, row_body, 0)
        pltpu.sync_copy(a, out_hbm.at[pl.ds(base, _TPW)])

    return k(eo, se, so, we, wo)


def kernel(hidden_states, W_router, W_gate, W_up, W_down):
    b, s, h = hidden_states.shape
    x = hidden_states.reshape(_T, _H)
    se2, so2, we2, wo2, loss = _router(x, W_router)
    se = se2.reshape(_T)
    so = so2.reshape(_T)
    we = we2.reshape(_T, 16)
    wo = wo2.reshape(_T, 16)
    buf = _dispatch(x, se, so)
    eo = _mlp(buf.reshape(_E, _CAPP, _H), W_gate, W_up, W_down)
    out = _combine(eo.reshape(_NROWS, _H), se, so, we, wo)
    return out.reshape(b, s, h), loss[0, 0]


# speedup vs baseline: 1.1476x; 1.1476x over previous
import functools

import jax
import jax.numpy as jnp
from jax import lax
from jax.experimental import pallas as pl
from jax.experimental.pallas import tpu as pltpu
from jax.experimental.pallas import tpu_sc as plsc

_H = 768
_F = 1024
_E = 64
_T = 2048
_CAP = 256
_CAPP = 264
_NROWS = _E * _CAPP
_AUX_COEF = 0.01
_Z_COEF = 0.001
_NW = 32
_TPW = _T // _NW


def _router_body(x_ref, wr_ref, se_ref, so_ref, we_ref, wo_ref, loss_ref):
    x = x_ref[...]
    wr = wr_ref[...]
    logits = lax.dot_general(x, wr, (((1,), (1,)), ((), ())),
                             preferred_element_type=jnp.float32)
    l3 = logits.reshape(16, 128, _E)
    m3 = jnp.max(l3, axis=2, keepdims=True)
    el = jnp.exp(l3 - m3)
    s3 = jnp.sum(el, axis=2, keepdims=True)
    probs = el / s3
    lse = m3[:, :, 0] + jnp.log(s3[:, :, 0])
    z_loss = _Z_COEF * jnp.mean(lse * lse)

    iota3 = lax.broadcasted_iota(jnp.int32, (16, 128, _E), 2)
    v1 = jnp.max(probs, axis=2, keepdims=True)
    i1 = jnp.min(jnp.where(probs == v1, iota3, _E), axis=2)
    oh1 = (iota3 == i1[:, :, None]).astype(jnp.float32)
    probs_m = jnp.where(oh1 > 0.0, -1.0, probs)
    v2 = jnp.max(probs_m, axis=2, keepdims=True)
    i2 = jnp.min(jnp.where(probs_m == v2, iota3, _E), axis=2)
    oh2 = (iota3 == i2[:, :, None]).astype(jnp.float32)

    denom = v1[:, :, 0] + v2[:, :, 0]
    w1 = v1[:, :, 0] / denom
    w2 = v2[:, :, 0] / denom

    oh12 = oh1 + oh2
    counts_e = jnp.sum(oh12, axis=(0, 1))
    f_e = counts_e / (2.0 * _T)
    p_e = jnp.mean(probs, axis=(0, 1))
    aux = _AUX_COEF * _E * jnp.sum(f_e * p_e)
    loss_ref[...] = jnp.broadcast_to(aux + z_loss, (1, 1))

    rr = lax.broadcasted_iota(jnp.int32, (128, 128), 0)
    cc = lax.broadcasted_iota(jnp.int32, (128, 128), 1)
    tril = (rr > cc).astype(jnp.float32)
    rr16 = lax.broadcasted_iota(jnp.int32, (16, 16), 0)
    cc16 = lax.broadcasted_iota(jnp.int32, (16, 16), 1)
    tril16 = (rr16 > cc16).astype(jnp.float32)
    bsums = jnp.sum(oh12, axis=1)
    offs = lax.dot_general(tril16, bsums, (((1,), (0,)), ((), ())),
                           preferred_element_type=jnp.float32)
    pos0_l, pos1_l = [], []
    for b in range(16):
        cx = lax.dot_general(tril, oh12[b], (((1,), (0,)), ((), ())),
                             preferred_element_type=jnp.float32)
        cx = cx + offs[b][None, :]
        pos0_l.append(jnp.sum(cx * oh1[b], axis=1))
        pos1_l.append(jnp.sum(cx * oh2[b], axis=1))
    pos0 = jnp.stack(pos0_l)
    pos1 = jnp.stack(pos1_l)
    keep0 = (pos0 < float(_CAP)).astype(jnp.float32)
    keep1 = (pos1 < float(_CAP)).astype(jnp.float32)
    slot0 = jnp.minimum(pos0, float(_CAP)).astype(jnp.int32)
    slot1 = jnp.minimum(pos1, float(_CAP)).astype(jnp.int32)
    se_ref[...] = i1 * _CAPP + slot0
    so_ref[...] = i2 * _CAPP + slot1
    we_ref[...] = jnp.broadcast_to((w1 * keep0)[:, :, None], (16, 128, 16))
    wo_ref[...] = jnp.broadcast_to((w2 * keep1)[:, :, None], (16, 128, 16))


def _router(x, w_router):
    return pl.pallas_call(
        _router_body,
        out_shape=(
            jax.ShapeDtypeStruct((16, 128), jnp.int32),
            jax.ShapeDtypeStruct((16, 128), jnp.int32),
            jax.ShapeDtypeStruct((16, 128, 16), jnp.float32),
            jax.ShapeDtypeStruct((16, 128, 16), jnp.float32),
            jax.ShapeDtypeStruct((1, 1), jnp.float32),
        ),
    )(x, w_router)


def _dispatch(x, se, so):
    mesh = plsc.VectorSubcoreMesh(core_axis_name="c", subcore_axis_name="s",
                                  num_cores=2, num_subcores=16)

    @functools.partial(
        pl.kernel,
        mesh=mesh,
        out_type=jax.ShapeDtypeStruct((_NROWS, _H), jnp.float32),
        scratch_types=[
            pltpu.VMEM((_TPW,), jnp.int32),
            pltpu.VMEM((_TPW,), jnp.int32),
            pltpu.VMEM((_TPW, _H), jnp.float32),
            pltpu.SemaphoreType.DMA,
        ],
    )
    def k(x_hbm, se_hbm, so_hbm, buf_hbm, idx_e, idx_o, rows, sem):
        wid = lax.axis_index("s") * 2 + lax.axis_index("c")
        base = wid * _TPW
        pltpu.sync_copy(x_hbm.at[pl.ds(base, _TPW)], rows)
        pltpu.sync_copy(se_hbm.at[pl.ds(base, _TPW)], idx_e)
        pltpu.sync_copy(so_hbm.at[pl.ds(base, _TPW)], idx_o)
        pltpu.async_copy(rows, buf_hbm.at[idx_e], sem).wait()
        pltpu.async_copy(rows, buf_hbm.at[idx_o], sem).wait()

    return k(x, se, so)


def _mlp_body(buf_ref, wg_ref, wu_ref, wd_ref, out_ref):
    xb = buf_ref[0].astype(jnp.bfloat16)
    wg = wg_ref[0].astype(jnp.bfloat16)
    wu = wu_ref[0].astype(jnp.bfloat16)
    g = jnp.dot(xb, wg, preferred_element_type=jnp.float32)
    u = jnp.dot(xb, wu, preferred_element_type=jnp.float32)
    h = (g / (1.0 + jnp.exp(-g))) * u
    wd = wd_ref[0].astype(jnp.bfloat16)
    out_ref[0] = jnp.dot(h.astype(jnp.bfloat16), wd,
                         preferred_element_type=jnp.float32)


def _mlp(buf3, w_gate, w_up, w_down):
    return pl.pallas_call(
        _mlp_body,
        grid=(_E,),
        in_specs=[
            pl.BlockSpec((1, _CAPP, _H), lambda e: (e, 0, 0)),
            pl.BlockSpec((1, _H, _F), lambda e: (e, 0, 0)),
            pl.BlockSpec((1, _H, _F), lambda e: (e, 0, 0)),
            pl.BlockSpec((1, _F, _H), lambda e: (e, 0, 0)),
        ],
        out_specs=pl.BlockSpec((1, _CAPP, _H), lambda e: (e, 0, 0)),
        out_shape=jax.ShapeDtypeStruct((_E, _CAPP, _H), jnp.float32),
        compiler_params=pltpu.CompilerParams(
            dimension_semantics=("arbitrary",)),
    )(buf3, w_gate, w_up, w_down)


def _combine(eo, se, so, we, wo):
    mesh = plsc.VectorSubcoreMesh(core_axis_name="c", subcore_axis_name="s",
                                  num_cores=2, num_subcores=16)
    @functools.partial(
        pl.kernel,
        mesh=mesh,
        out_type=jax.ShapeDtypeStruct((_T, _H), jnp.float32),
        scratch_types=[
            pltpu.VMEM((_TPW,), jnp.int32),
            pltpu.VMEM((_TPW,), jnp.int32),
            pltpu.VMEM((_TPW, 16), jnp.float32),
            pltpu.VMEM((_TPW, 16), jnp.float32),
            pltpu.VMEM((_TPW, _H), jnp.float32),
            pltpu.VMEM((_TPW, _H), jnp.float32),
            [pltpu.SemaphoreType.DMA] * 4,
            pltpu.SemaphoreType.DMA,
        ],
    )
    def k(eo_hbm, se_hbm, so_hbm, we_hbm, wo_hbm, out_hbm,
          idx_e, idx_o, w0v, w1v, a, b, gsems, osem):
        wid = lax.axis_index("s") * 2 + lax.axis_index("c")
        base = wid * _TPW
        gsz = _TPW // 4
        pltpu.sync_copy(se_hbm.at[pl.ds(base, _TPW)], idx_e)
        pltpu.sync_copy(so_hbm.at[pl.ds(base, _TPW)], idx_o)
        pltpu.sync_copy(we_hbm.at[pl.ds(base, _TPW)], w0v)
        pltpu.sync_copy(wo_hbm.at[pl.ds(base, _TPW)], w1v)
        cps = []
        for g in range(4):
            gs = pl.ds(g * gsz, gsz)
            cps.append((
                pltpu.async_copy(eo_hbm.at[idx_e.at[gs]], a.at[gs], gsems[g]),
                pltpu.async_copy(eo_hbm.at[idx_o.at[gs]], b.at[gs], gsems[g]),
            ))
        outcps = []
        for g in range(4):
            cps[g][0].wait()
            cps[g][1].wait()

            def row_body(t, carry):
                w0 = w0v[t, :]
                w1 = w1v[t, :]
                for ci in range(_H // 16):
                    sl = pl.ds(ci * 16, 16)
                    a[t, sl] = a[t, sl] * w0 + b[t, sl] * w1
                return carry

            lax.fori_loop(g * gsz, (g + 1) * gsz, row_body, 0)
            outcps.append(pltpu.async_copy(
                a.at[pl.ds(g * gsz, gsz)],
                out_hbm.at[pl.ds(base + g * gsz, gsz)], osem))
        for cp in outcps:
            cp.wait()

    return k(eo, se, so, we, wo)


def kernel(hidden_states, W_router, W_gate, W_up, W_down):
    b, s, h = hidden_states.shape
    x = hidden_states.reshape(_T, _H)
    se2, so2, we2, wo2, loss = _router(x, W_router)
    se = se2.reshape(_T)
    so = so2.reshape(_T)
    we = we2.reshape(_T, 16)
    wo = wo2.reshape(_T, 16)
    buf = _dispatch(x, se, so)
    eo = _mlp(buf.reshape(_E, _CAPP, _H), W_gate, W_up, W_down)
    out = _combine(eo.reshape(_NROWS, _H), se, so, we, wo)
    return out.reshape(b, s, h), loss[0, 0]
